# initial kernel scaffold (unmeasured)
import jax
import jax.numpy as jnp
from jax import lax
from jax.experimental import pallas as pl
from jax.experimental.pallas import tpu as pltpu

N_DEV = 4
N_LAYERS = 3


def kernel(x, Win0, Wout0, Win1, Wout1, Win2, Wout2):
    b, d_loc = x.shape
    _, h = Win0.shape

    def body(x_ref, win0_ref, wout0_ref, win1_ref, wout1_ref,
             win2_ref, wout2_ref, out_ref,
             partial_buf, recv_buf, send_sems, recv_sems):
        my = lax.axis_index("i")
        wins = [win0_ref, win1_ref, win2_ref]
        wouts = [wout0_ref, wout1_ref, wout2_ref]

        x_cur = x_ref[...]
        for l in range(N_LAYERS):
            partial_buf[...] = jnp.dot(
                x_cur, wins[l][...], preferred_element_type=jnp.float32
            )
            rdmas = []
            for k in range(N_DEV - 1):
                peer = lax.rem(my + 1 + k, N_DEV)
                slot = N_DEV - 2 - k
                rdma = pltpu.make_async_remote_copy(
                    src_ref=partial_buf,
                    dst_ref=recv_buf.at[l, slot],
                    send_sem=send_sems.at[l, k],
                    recv_sem=recv_sems.at[l, slot],
                    device_id=(peer,),
                    device_id_type=pl.DeviceIdType.MESH,
                )
                rdma.start()
                rdmas.append(rdma)
            for rdma in rdmas:
                rdma.wait()
            total = (
                partial_buf[...]
                + recv_buf[l, 0]
                + recv_buf[l, 1]
                + recv_buf[l, 2]
            )
            hact = jnp.maximum(total, 0.0)
            x_cur = jnp.dot(
                hact, wouts[l][...], preferred_element_type=jnp.float32
            )
        out_ref[...] = x_cur

    return pl.pallas_call(
        body,
        out_shape=jax.ShapeDtypeStruct((b, d_loc), jnp.float32),
        in_specs=[pl.BlockSpec(memory_space=pltpu.VMEM)] * 7,
        out_specs=pl.BlockSpec(memory_space=pltpu.VMEM),
        scratch_shapes=[
            pltpu.VMEM((b, h), jnp.float32),
            pltpu.VMEM((N_LAYERS, N_DEV - 1, b, h), jnp.float32),
            pltpu.SemaphoreType.DMA((N_LAYERS, N_DEV - 1)),
            pltpu.SemaphoreType.DMA((N_LAYERS, N_DEV - 1)),
        ],
        compiler_params=pltpu.CompilerParams(collective_id=0),
    )(x, Win0, Wout0, Win1, Wout1, Win2, Wout2)


# baseline (device time: 54908 ns/iter reference)
import jax
import jax.numpy as jnp
from jax import lax
from jax.experimental import pallas as pl
from jax.experimental.pallas import tpu as pltpu

N_DEV = 4
N_LAYERS = 3


def kernel(x, Win0, Wout0, Win1, Wout1, Win2, Wout2):
    b, d_loc = x.shape
    _, h = Win0.shape

    def body(x_ref, win0_ref, wout0_ref, win1_ref, wout1_ref,
             win2_ref, wout2_ref, out_ref,
             partial_buf, recv_buf, send_sems, recv_sems):
        my = lax.axis_index("i")
        wins = [win0_ref, win1_ref, win2_ref]
        wouts = [wout0_ref, wout1_ref, wout2_ref]

        x_cur = x_ref[...]
        for l in range(N_LAYERS):
            partial_buf[...] = jnp.dot(
                x_cur, wins[l][...], preferred_element_type=jnp.float32
            )
            rdmas = []
            for k in range(N_DEV - 1):
                peer = lax.rem(my + 1 + k, N_DEV)
                slot = N_DEV - 2 - k
                rdma = pltpu.make_async_remote_copy(
                    src_ref=partial_buf,
                    dst_ref=recv_buf.at[l, slot],
                    send_sem=send_sems.at[l, k],
                    recv_sem=recv_sems.at[l, slot],
                    device_id=(peer,),
                    device_id_type=pl.DeviceIdType.MESH,
                )
                rdma.start()
                rdmas.append(rdma)
            for rdma in rdmas:
                rdma.wait()
            total = (
                partial_buf[...]
                + recv_buf[l, 0]
                + recv_buf[l, 1]
                + recv_buf[l, 2]
            )
            hact = jnp.maximum(total, 0.0)
            x_cur = jnp.dot(
                hact, wouts[l][...], preferred_element_type=jnp.float32
            )
        out_ref[...] = x_cur

    return pl.pallas_call(
        body,
        out_shape=jax.ShapeDtypeStruct((b, d_loc), jnp.float32),
        in_specs=[pl.BlockSpec(memory_space=pltpu.VMEM)] * 7,
        out_specs=pl.BlockSpec(memory_space=pltpu.VMEM),
        scratch_shapes=[
            pltpu.VMEM((b, h), jnp.float32),
            pltpu.VMEM((N_LAYERS, N_DEV - 1, b, h), jnp.float32),
            pltpu.SemaphoreType.DMA((N_LAYERS, N_DEV - 1)),
            pltpu.SemaphoreType.DMA((N_LAYERS, N_DEV - 1)),
        ],
    )(x, Win0, Wout0, Win1, Wout1, Win2, Wout2)


# device time: 45075 ns/iter; 1.2181x vs baseline; 1.2181x over previous
import jax
import jax.numpy as jnp
from jax import lax
from jax.experimental import pallas as pl
from jax.experimental.pallas import tpu as pltpu

N_DEV = 4
N_LAYERS = 3
QH = 128


def kernel(x, Win0, Wout0, Win1, Wout1, Win2, Wout2):
    b, d_loc = x.shape
    _, h = Win0.shape

    def body(x_ref, win0_ref, wout0_ref, win1_ref, wout1_ref,
             win2_ref, wout2_ref, out_ref,
             pq_buf, rs_recv, hfull,
             rs_send_sems, rs_recv_sems, ag_send_sems, ag_recv_sems):
        my = lax.axis_index("i")
        wins = [win0_ref, win1_ref, win2_ref]
        wouts = [wout0_ref, wout1_ref, wout2_ref]

        x_cur = x_ref[...]
        for l in range(N_LAYERS):
            win = wins[l]
            wout = wouts[l]

            rs_rdmas = []
            for k in range(N_DEV - 1):
                peer = lax.rem(my + 1 + k, N_DEV)
                pq_buf[k] = jnp.dot(
                    x_cur, win[:, pl.ds(peer * QH, QH)],
                    preferred_element_type=jnp.float32,
                )
                rdma = pltpu.make_async_remote_copy(
                    src_ref=pq_buf.at[k],
                    dst_ref=rs_recv.at[N_DEV - 2 - k],
                    send_sem=rs_send_sems.at[k],
                    recv_sem=rs_recv_sems.at[N_DEV - 2 - k],
                    device_id=(peer,),
                    device_id_type=pl.DeviceIdType.MESH,
                )
                rdma.start()
                rs_rdmas.append(rdma)

            own = jnp.dot(
                x_cur, win[:, pl.ds(my * QH, QH)],
                preferred_element_type=jnp.float32,
            )
            for rdma in rs_rdmas:
                rdma.wait()
            hq = jnp.maximum(
                own + rs_recv[0] + rs_recv[1] + rs_recv[2], 0.0
            )
            hfull[my] = hq

            ag_rdmas = []
            for k in range(N_DEV - 1):
                peer = lax.rem(my + 1 + k, N_DEV)
                rdma = pltpu.make_async_remote_copy(
                    src_ref=hfull.at[my],
                    dst_ref=hfull.at[my],
                    send_sem=ag_send_sems.at[k],
                    recv_sem=ag_recv_sems.at[N_DEV - 2 - k],
                    device_id=(peer,),
                    device_id_type=pl.DeviceIdType.MESH,
                )
                rdma.start()
                ag_rdmas.append(rdma)

            acc = jnp.dot(
                hq, wout[pl.ds(my * QH, QH), :],
                preferred_element_type=jnp.float32,
            )
            for k in range(N_DEV - 1):
                ag_rdmas[k].wait()
                qidx = lax.rem(my + N_DEV - 1 - k, N_DEV)
                acc = acc + jnp.dot(
                    hfull[qidx], wout[pl.ds(qidx * QH, QH), :],
                    preferred_element_type=jnp.float32,
                )
            x_cur = acc
        out_ref[...] = x_cur

    return pl.pallas_call(
        body,
        out_shape=jax.ShapeDtypeStruct((b, d_loc), jnp.float32),
        in_specs=[pl.BlockSpec(memory_space=pltpu.VMEM)] * 7,
        out_specs=pl.BlockSpec(memory_space=pltpu.VMEM),
        scratch_shapes=[
            pltpu.VMEM((N_DEV - 1, b, QH), jnp.float32),
            pltpu.VMEM((N_DEV - 1, b, QH), jnp.float32),
            pltpu.VMEM((N_DEV, b, QH), jnp.float32),
            pltpu.SemaphoreType.DMA((N_DEV - 1,)),
            pltpu.SemaphoreType.DMA((N_DEV - 1,)),
            pltpu.SemaphoreType.DMA((N_DEV - 1,)),
            pltpu.SemaphoreType.DMA((N_DEV - 1,)),
        ],
    )(x, Win0, Wout0, Win1, Wout1, Win2, Wout2)


# device time: 38162 ns/iter; 1.4388x vs baseline; 1.1811x over previous
import jax
import jax.numpy as jnp
from jax import lax
from jax.experimental import pallas as pl
from jax.experimental.pallas import tpu as pltpu

N_DEV = 4
N_LAYERS = 3
QH = 128
C = 2
R = 128


def kernel(x, Win0, Wout0, Win1, Wout1, Win2, Wout2):
    b, d_loc = x.shape

    def body(x_ref, win0_ref, wout0_ref, win1_ref, wout1_ref,
             win2_ref, wout2_ref, out_ref,
             pq_buf, rs_recv, hfull,
             rs_send_sems, rs_recv_sems, ag_send_sems, ag_recv_sems):
        my = lax.axis_index("i")
        wins = [win0_ref, win1_ref, win2_ref]
        wouts = [wout0_ref, wout1_ref, wout2_ref]

        def start_rs(c, l, x_cur):
            win = wins[l]
            rdmas = []
            for k in range(N_DEV - 1):
                peer = lax.rem(my + 1 + k, N_DEV)
                pq_buf[c, k] = jnp.dot(
                    x_cur, win[:, pl.ds(peer * QH, QH)],
                    preferred_element_type=jnp.float32,
                )
                rdma = pltpu.make_async_remote_copy(
                    src_ref=pq_buf.at[c, k],
                    dst_ref=rs_recv.at[c, N_DEV - 2 - k],
                    send_sem=rs_send_sems.at[c, k],
                    recv_sem=rs_recv_sems.at[c, N_DEV - 2 - k],
                    device_id=(peer,),
                    device_id_type=pl.DeviceIdType.MESH,
                )
                rdma.start()
                rdmas.append(rdma)
            own = jnp.dot(
                x_cur, win[:, pl.ds(my * QH, QH)],
                preferred_element_type=jnp.float32,
            )
            return rdmas, own

        def reduce_and_start_ag(c, l, rs_rdmas, own):
            for rdma in rs_rdmas:
                rdma.wait()
            hq = jnp.maximum(
                own + rs_recv[c, 0] + rs_recv[c, 1] + rs_recv[c, 2], 0.0
            )
            hfull[c, my] = hq
            ag_rdmas = []
            for k in range(N_DEV - 1):
                peer = lax.rem(my + 1 + k, N_DEV)
                rdma = pltpu.make_async_remote_copy(
                    src_ref=hfull.at[c, my],
                    dst_ref=hfull.at[c, my],
                    send_sem=ag_send_sems.at[c, k],
                    recv_sem=ag_recv_sems.at[c, N_DEV - 2 - k],
                    device_id=(peer,),
                    device_id_type=pl.DeviceIdType.MESH,
                )
                rdma.start()
                ag_rdmas.append(rdma)
            acc = jnp.dot(
                hq, wouts[l][pl.ds(my * QH, QH), :],
                preferred_element_type=jnp.float32,
            )
            return ag_rdmas, acc

        def finish_ag(c, l, ag_rdmas, acc):
            for k in (0, 2, 1):
                ag_rdmas[k].wait()
                qidx = lax.rem(my + N_DEV - 1 - k, N_DEV)
                acc = acc + jnp.dot(
                    hfull[c, qidx], wouts[l][pl.ds(qidx * QH, QH), :],
                    preferred_element_type=jnp.float32,
                )
            return acc

        xs = [x_ref[c * R:(c + 1) * R, :] for c in range(C)]
        st = [start_rs(c, 0, xs[c]) for c in range(C)]
        for l in range(N_LAYERS):
            mids = [reduce_and_start_ag(c, l, *st[c]) for c in range(C)]
            for c in range(C):
                xc = finish_ag(c, l, *mids[c])
                if l < N_LAYERS - 1:
                    st[c] = start_rs(c, l + 1, xc)
                else:
                    out_ref[c * R:(c + 1) * R, :] = xc

    return pl.pallas_call(
        body,
        out_shape=jax.ShapeDtypeStruct((b, d_loc), jnp.float32),
        in_specs=[pl.BlockSpec(memory_space=pltpu.VMEM)] * 7,
        out_specs=pl.BlockSpec(memory_space=pltpu.VMEM),
        scratch_shapes=[
            pltpu.VMEM((C, N_DEV - 1, R, QH), jnp.float32),
            pltpu.VMEM((C, N_DEV - 1, R, QH), jnp.float32),
            pltpu.VMEM((C, N_DEV, R, QH), jnp.float32),
            pltpu.SemaphoreType.DMA((C, N_DEV - 1)),
            pltpu.SemaphoreType.DMA((C, N_DEV - 1)),
            pltpu.SemaphoreType.DMA((C, N_DEV - 1)),
            pltpu.SemaphoreType.DMA((C, N_DEV - 1)),
        ],
    )(x, Win0, Wout0, Win1, Wout1, Win2, Wout2)


# device time: 33227 ns/iter; 1.6525x vs baseline; 1.1485x over previous
import jax
import jax.numpy as jnp
from jax import lax
from jax.experimental import pallas as pl
from jax.experimental.pallas import tpu as pltpu

N_DEV = 4
N_LAYERS = 3
QH = 128
C = 2
R = 128


def kernel(x, Win0, Wout0, Win1, Wout1, Win2, Wout2):
    b, d_loc = x.shape

    def body(x_ref, win0_ref, wout0_ref, win1_ref, wout1_ref,
             win2_ref, wout2_ref, out_ref,
             pq_buf, rs_recv, hfull,
             rs_send_sems, rs_recv_sems, ag_send_sems, ag_recv_sems):
        my = lax.axis_index("i")
        wins = [win0_ref, win1_ref, win2_ref]
        wouts = [wout0_ref, wout1_ref, wout2_ref]

        def start_rs(c, l, x_cur):
            win = wins[l]
            rdmas = []
            for k in range(N_DEV - 1):
                peer = lax.rem(my + 1 + k, N_DEV)
                pq_buf[c, k] = jnp.dot(
                    x_cur, win[:, pl.ds(peer * QH, QH)],
                    preferred_element_type=jnp.float32,
                ).astype(jnp.bfloat16)
                rdma = pltpu.make_async_remote_copy(
                    src_ref=pq_buf.at[c, k],
                    dst_ref=rs_recv.at[c, N_DEV - 2 - k],
                    send_sem=rs_send_sems.at[c, k],
                    recv_sem=rs_recv_sems.at[c, N_DEV - 2 - k],
                    device_id=(peer,),
                    device_id_type=pl.DeviceIdType.MESH,
                )
                rdma.start()
                rdmas.append(rdma)
            own = jnp.dot(
                x_cur, win[:, pl.ds(my * QH, QH)],
                preferred_element_type=jnp.float32,
            )
            return rdmas, own

        def reduce_and_start_ag(c, l, rs_rdmas, own):
            for rdma in rs_rdmas:
                rdma.wait()
            hq = jnp.maximum(
                own
                + rs_recv[c, 0].astype(jnp.float32)
                + rs_recv[c, 1].astype(jnp.float32)
                + rs_recv[c, 2].astype(jnp.float32),
                0.0,
            )
            hfull[c, my] = hq.astype(jnp.bfloat16)
            ag_rdmas = []
            for k in range(N_DEV - 1):
                peer = lax.rem(my + 1 + k, N_DEV)
                rdma = pltpu.make_async_remote_copy(
                    src_ref=hfull.at[c, my],
                    dst_ref=hfull.at[c, my],
                    send_sem=ag_send_sems.at[c, k],
                    recv_sem=ag_recv_sems.at[c, N_DEV - 2 - k],
                    device_id=(peer,),
                    device_id_type=pl.DeviceIdType.MESH,
                )
                rdma.start()
                ag_rdmas.append(rdma)
            acc = jnp.dot(
                hq, wouts[l][pl.ds(my * QH, QH), :],
                preferred_element_type=jnp.float32,
            )
            return ag_rdmas, acc

        def finish_ag(c, l, ag_rdmas, acc):
            for k in (0, 2, 1):
                ag_rdmas[k].wait()
                qidx = lax.rem(my + N_DEV - 1 - k, N_DEV)
                acc = acc + jnp.dot(
                    hfull[c, qidx].astype(jnp.float32),
                    wouts[l][pl.ds(qidx * QH, QH), :],
                    preferred_element_type=jnp.float32,
                )
            return acc

        xs = [x_ref[c * R:(c + 1) * R, :] for c in range(C)]
        st = [start_rs(c, 0, xs[c]) for c in range(C)]
        for l in range(N_LAYERS):
            mids = [reduce_and_start_ag(c, l, *st[c]) for c in range(C)]
            for c in range(C):
                xc = finish_ag(c, l, *mids[c])
                if l < N_LAYERS - 1:
                    st[c] = start_rs(c, l + 1, xc)
                else:
                    out_ref[c * R:(c + 1) * R, :] = xc

    return pl.pallas_call(
        body,
        out_shape=jax.ShapeDtypeStruct((b, d_loc), jnp.float32),
        in_specs=[pl.BlockSpec(memory_space=pltpu.VMEM)] * 7,
        out_specs=pl.BlockSpec(memory_space=pltpu.VMEM),
        scratch_shapes=[
            pltpu.VMEM((C, N_DEV - 1, R, QH), jnp.bfloat16),
            pltpu.VMEM((C, N_DEV - 1, R, QH), jnp.bfloat16),
            pltpu.VMEM((C, N_DEV, R, QH), jnp.bfloat16),
            pltpu.SemaphoreType.DMA((C, N_DEV - 1)),
            pltpu.SemaphoreType.DMA((C, N_DEV - 1)),
            pltpu.SemaphoreType.DMA((C, N_DEV - 1)),
            pltpu.SemaphoreType.DMA((C, N_DEV - 1)),
        ],
    )(x, Win0, Wout0, Win1, Wout1, Win2, Wout2)


# device time: 32541 ns/iter; 1.6873x vs baseline; 1.0211x over previous
import jax
import jax.numpy as jnp
from jax import lax
from jax.experimental import pallas as pl
from jax.experimental.pallas import tpu as pltpu

N_DEV = 4
N_LAYERS = 3
QH = 128
C = 4
R = 64


def kernel(x, Win0, Wout0, Win1, Wout1, Win2, Wout2):
    b, d_loc = x.shape

    def body(x_ref, win0_ref, wout0_ref, win1_ref, wout1_ref,
             win2_ref, wout2_ref, out_ref,
             pq_buf, rs_recv, hfull,
             rs_send_sems, rs_recv_sems, ag_send_sems, ag_recv_sems):
        my = lax.axis_index("i")
        wins = [win0_ref, win1_ref, win2_ref]
        wouts = [wout0_ref, wout1_ref, wout2_ref]

        def start_rs(c, l, x_cur):
            win = wins[l]
            xb = x_cur.astype(jnp.bfloat16)
            rdmas = []
            for k in range(N_DEV - 1):
                peer = lax.rem(my + 1 + k, N_DEV)
                pq_buf[c, k] = jnp.dot(
                    xb, win[:, pl.ds(peer * QH, QH)].astype(jnp.bfloat16),
                    preferred_element_type=jnp.float32,
                ).astype(jnp.bfloat16)
                rdma = pltpu.make_async_remote_copy(
                    src_ref=pq_buf.at[c, k],
                    dst_ref=rs_recv.at[c, N_DEV - 2 - k],
                    send_sem=rs_send_sems.at[c, k],
                    recv_sem=rs_recv_sems.at[c, N_DEV - 2 - k],
                    device_id=(peer,),
                    device_id_type=pl.DeviceIdType.MESH,
                )
                rdma.start()
                rdmas.append(rdma)
            own = jnp.dot(
                xb, win[:, pl.ds(my * QH, QH)].astype(jnp.bfloat16),
                preferred_element_type=jnp.float32,
            )
            return rdmas, own

        def reduce_and_start_ag(c, l, rs_rdmas, own):
            for rdma in rs_rdmas:
                rdma.wait()
            hq = jnp.maximum(
                own
                + rs_recv[c, 0].astype(jnp.float32)
                + rs_recv[c, 1].astype(jnp.float32)
                + rs_recv[c, 2].astype(jnp.float32),
                0.0,
            )
            hfull[c, my] = hq.astype(jnp.bfloat16)
            ag_rdmas = []
            for k in range(N_DEV - 1):
                peer = lax.rem(my + 1 + k, N_DEV)
                rdma = pltpu.make_async_remote_copy(
                    src_ref=hfull.at[c, my],
                    dst_ref=hfull.at[c, my],
                    send_sem=ag_send_sems.at[c, k],
                    recv_sem=ag_recv_sems.at[c, N_DEV - 2 - k],
                    device_id=(peer,),
                    device_id_type=pl.DeviceIdType.MESH,
                )
                rdma.start()
                ag_rdmas.append(rdma)
            acc = jnp.dot(
                hq.astype(jnp.bfloat16),
                wouts[l][pl.ds(my * QH, QH), :].astype(jnp.bfloat16),
                preferred_element_type=jnp.float32,
            )
            return ag_rdmas, acc

        def finish_ag(c, l, ag_rdmas, acc):
            for k in (0, 2, 1):
                ag_rdmas[k].wait()
                qidx = lax.rem(my + N_DEV - 1 - k, N_DEV)
                acc = acc + jnp.dot(
                    hfull[c, qidx],
                    wouts[l][pl.ds(qidx * QH, QH), :].astype(jnp.bfloat16),
                    preferred_element_type=jnp.float32,
                )
            return acc

        xs = [x_ref[c * R:(c + 1) * R, :] for c in range(C)]
        st = [start_rs(c, 0, xs[c]) for c in range(C)]
        for l in range(N_LAYERS):
            mids = [reduce_and_start_ag(c, l, *st[c]) for c in range(C)]
            for c in range(C):
                xc = finish_ag(c, l, *mids[c])
                if l < N_LAYERS - 1:
                    st[c] = start_rs(c, l + 1, xc)
                else:
                    out_ref[c * R:(c + 1) * R, :] = xc

    return pl.pallas_call(
        body,
        out_shape=jax.ShapeDtypeStruct((b, d_loc), jnp.float32),
        in_specs=[pl.BlockSpec(memory_space=pltpu.VMEM)] * 7,
        out_specs=pl.BlockSpec(memory_space=pltpu.VMEM),
        scratch_shapes=[
            pltpu.VMEM((C, N_DEV - 1, R, QH), jnp.bfloat16),
            pltpu.VMEM((C, N_DEV - 1, R, QH), jnp.bfloat16),
            pltpu.VMEM((C, N_DEV, R, QH), jnp.bfloat16),
            pltpu.SemaphoreType.DMA((C, N_DEV - 1)),
            pltpu.SemaphoreType.DMA((C, N_DEV - 1)),
            pltpu.SemaphoreType.DMA((C, N_DEV - 1)),
            pltpu.SemaphoreType.DMA((C, N_DEV - 1)),
        ],
    )(x, Win0, Wout0, Win1, Wout1, Win2, Wout2)


# device time: 31269 ns/iter; 1.7560x vs baseline; 1.0407x over previous
import jax
import jax.numpy as jnp
from jax import lax
from jax.experimental import pallas as pl
from jax.experimental.pallas import tpu as pltpu

N_DEV = 4
N_LAYERS = 3
H = 512
QH = 128
C = 4
R = 64


def kernel(x, Win0, Wout0, Win1, Wout1, Win2, Wout2):
    b, d_loc = x.shape

    def body(x_ref, win0_ref, wout0_ref, win1_ref, wout1_ref,
             win2_ref, wout2_ref, out_ref,
             win_bf, wout_bf, pq_buf, rs_recv, hfull,
             rs_send_sems, rs_recv_sems, ag_send_sems, ag_recv_sems):
        my = lax.axis_index("i")
        wins = [win0_ref, win1_ref, win2_ref]
        wouts = [wout0_ref, wout1_ref, wout2_ref]

        for l in range(N_LAYERS):
            win_bf[l] = wins[l][...].astype(jnp.bfloat16)
            wout_bf[l] = wouts[l][...].astype(jnp.bfloat16)

        def start_rs(c, l, x_cur):
            partial = jnp.dot(
                x_cur.astype(jnp.bfloat16), win_bf[l],
                preferred_element_type=jnp.float32,
            )
            pbf = partial.astype(jnp.bfloat16)
            for q in range(N_DEV):
                pq_buf[c, q] = pbf[:, q * QH:(q + 1) * QH]
            rdmas = []
            for k in range(N_DEV - 1):
                peer = lax.rem(my + 1 + k, N_DEV)
                rdma = pltpu.make_async_remote_copy(
                    src_ref=pq_buf.at[c, peer],
                    dst_ref=rs_recv.at[c, N_DEV - 2 - k],
                    send_sem=rs_send_sems.at[c, k],
                    recv_sem=rs_recv_sems.at[c, N_DEV - 2 - k],
                    device_id=(peer,),
                    device_id_type=pl.DeviceIdType.MESH,
                )
                rdma.start()
                rdmas.append(rdma)
            own = pq_buf[c, my].astype(jnp.float32)
            return rdmas, own

        def reduce_and_start_ag(c, l, rs_rdmas, own):
            for rdma in rs_rdmas:
                rdma.wait()
            hq = jnp.maximum(
                own
                + rs_recv[c, 0].astype(jnp.float32)
                + rs_recv[c, 1].astype(jnp.float32)
                + rs_recv[c, 2].astype(jnp.float32),
                0.0,
            )
            hfull[c, my] = hq.astype(jnp.bfloat16)
            ag_rdmas = []
            for k in range(N_DEV - 1):
                peer = lax.rem(my + 1 + k, N_DEV)
                rdma = pltpu.make_async_remote_copy(
                    src_ref=hfull.at[c, my],
                    dst_ref=hfull.at[c, my],
                    send_sem=ag_send_sems.at[c, k],
                    recv_sem=ag_recv_sems.at[c, N_DEV - 2 - k],
                    device_id=(peer,),
                    device_id_type=pl.DeviceIdType.MESH,
                )
                rdma.start()
                ag_rdmas.append(rdma)
            return ag_rdmas

        def finish_ag(c, l, ag_rdmas):
            for rdma in ag_rdmas:
                rdma.wait()
            hcat = jnp.concatenate(
                [hfull[c, q] for q in range(N_DEV)], axis=1
            )
            return jnp.dot(
                hcat, wout_bf[l], preferred_element_type=jnp.float32
            )

        xs = [x_ref[c * R:(c + 1) * R, :] for c in range(C)]
        st = [start_rs(c, 0, xs[c]) for c in range(C)]
        for l in range(N_LAYERS):
            mids = [reduce_and_start_ag(c, l, *st[c]) for c in range(C)]
            for c in range(C):
                xc = finish_ag(c, l, mids[c])
                if l < N_LAYERS - 1:
                    st[c] = start_rs(c, l + 1, xc)
                else:
                    out_ref[c * R:(c + 1) * R, :] = xc

    return pl.pallas_call(
        body,
        out_shape=jax.ShapeDtypeStruct((b, d_loc), jnp.float32),
        in_specs=[pl.BlockSpec(memory_space=pltpu.VMEM)] * 7,
        out_specs=pl.BlockSpec(memory_space=pltpu.VMEM),
        scratch_shapes=[
            pltpu.VMEM((N_LAYERS, d_loc, H), jnp.bfloat16),
            pltpu.VMEM((N_LAYERS, H, d_loc), jnp.bfloat16),
            pltpu.VMEM((C, N_DEV, R, QH), jnp.bfloat16),
            pltpu.VMEM((C, N_DEV - 1, R, QH), jnp.bfloat16),
            pltpu.VMEM((C, N_DEV, R, QH), jnp.bfloat16),
            pltpu.SemaphoreType.DMA((C, N_DEV - 1)),
            pltpu.SemaphoreType.DMA((C, N_DEV - 1)),
            pltpu.SemaphoreType.DMA((C, N_DEV - 1)),
            pltpu.SemaphoreType.DMA((C, N_DEV - 1)),
        ],
    )(x, Win0, Wout0, Win1, Wout1, Win2, Wout2)


# device time: 28917 ns/iter; 1.8988x vs baseline; 1.0813x over previous
import jax
import jax.numpy as jnp
from jax import lax
from jax.experimental import pallas as pl
from jax.experimental.pallas import tpu as pltpu

N_DEV = 4
N_LAYERS = 3
H = 512
QH = 128
C = 4
R = 64


def kernel(x, Win0, Wout0, Win1, Wout1, Win2, Wout2):
    b, d_loc = x.shape

    def body(x_ref, win0_ref, wout0_ref, win1_ref, wout1_ref,
             win2_ref, wout2_ref, out_ref,
             win_bf, wout_bf, pq_buf, rs_recv, hfull,
             rs_send_sems, rs_recv_sems, ag_send_sems, ag_recv_sems):
        my = lax.axis_index("i")
        wins = [win0_ref, win1_ref, win2_ref]
        wouts = [wout0_ref, wout1_ref, wout2_ref]

        barrier_sem = pltpu.get_barrier_semaphore()
        for k in range(N_DEV - 1):
            pl.semaphore_signal(
                barrier_sem, inc=1,
                device_id=(lax.rem(my + 1 + k, N_DEV),),
                device_id_type=pl.DeviceIdType.MESH,
            )
        pl.semaphore_wait(barrier_sem, N_DEV - 1)

        for l in range(N_LAYERS):
            win_bf[l] = wins[l][...].astype(jnp.bfloat16)
            wout_bf[l] = wouts[l][...].astype(jnp.bfloat16)

        def start_rs(c, l, x_cur):
            partial = jnp.dot(
                x_cur.astype(jnp.bfloat16), win_bf[l],
                preferred_element_type=jnp.float32,
            )
            pbf = partial.astype(jnp.bfloat16)
            for q in range(N_DEV):
                pq_buf[c, q] = pbf[:, q * QH:(q + 1) * QH]
            rdmas = []
            for k in range(N_DEV - 1):
                peer = lax.rem(my + 1 + k, N_DEV)
                rdma = pltpu.make_async_remote_copy(
                    src_ref=pq_buf.at[c, peer],
                    dst_ref=rs_recv.at[c, N_DEV - 2 - k],
                    send_sem=rs_send_sems.at[c, k],
                    recv_sem=rs_recv_sems.at[c, N_DEV - 2 - k],
                    device_id=(peer,),
                    device_id_type=pl.DeviceIdType.MESH,
                )
                rdma.start()
                rdmas.append(rdma)
            own = pq_buf[c, my].astype(jnp.float32)
            return rdmas, own

        def reduce_and_start_ag(c, l, rs_rdmas, own):
            for rdma in rs_rdmas:
                rdma.wait()
            hq = jnp.maximum(
                own
                + rs_recv[c, 0].astype(jnp.float32)
                + rs_recv[c, 1].astype(jnp.float32)
                + rs_recv[c, 2].astype(jnp.float32),
                0.0,
            )
            hfull[c, my] = hq.astype(jnp.bfloat16)
            ag_rdmas = []
            for k in range(N_DEV - 1):
                peer = lax.rem(my + 1 + k, N_DEV)
                rdma = pltpu.make_async_remote_copy(
                    src_ref=hfull.at[c, my],
                    dst_ref=hfull.at[c, my],
                    send_sem=ag_send_sems.at[c, k],
                    recv_sem=ag_recv_sems.at[c, N_DEV - 2 - k],
                    device_id=(peer,),
                    device_id_type=pl.DeviceIdType.MESH,
                )
                rdma.start()
                ag_rdmas.append(rdma)
            return ag_rdmas

        def finish_ag(c, l, ag_rdmas):
            for rdma in ag_rdmas:
                rdma.wait()
            hcat = jnp.concatenate(
                [hfull[c, q] for q in range(N_DEV)], axis=1
            )
            return jnp.dot(
                hcat, wout_bf[l], preferred_element_type=jnp.float32
            )

        xs = [x_ref[c * R:(c + 1) * R, :] for c in range(C)]
        st = [start_rs(c, 0, xs[c]) for c in range(C)]
        for l in range(N_LAYERS):
            mids = [reduce_and_start_ag(c, l, *st[c]) for c in range(C)]
            for c in range(C):
                xc = finish_ag(c, l, mids[c])
                if l < N_LAYERS - 1:
                    st[c] = start_rs(c, l + 1, xc)
                else:
                    out_ref[c * R:(c + 1) * R, :] = xc

    return pl.pallas_call(
        body,
        out_shape=jax.ShapeDtypeStruct((b, d_loc), jnp.float32),
        in_specs=[pl.BlockSpec(memory_space=pltpu.VMEM)] * 7,
        out_specs=pl.BlockSpec(memory_space=pltpu.VMEM),
        scratch_shapes=[
            pltpu.VMEM((N_LAYERS, d_loc, H), jnp.bfloat16),
            pltpu.VMEM((N_LAYERS, H, d_loc), jnp.bfloat16),
            pltpu.VMEM((C, N_DEV, R, QH), jnp.bfloat16),
            pltpu.VMEM((C, N_DEV - 1, R, QH), jnp.bfloat16),
            pltpu.VMEM((C, N_DEV, R, QH), jnp.bfloat16),
            pltpu.SemaphoreType.DMA((C, N_DEV - 1)),
            pltpu.SemaphoreType.DMA((C, N_DEV - 1)),
            pltpu.SemaphoreType.DMA((C, N_DEV - 1)),
            pltpu.SemaphoreType.DMA((C, N_DEV - 1)),
        ],
        compiler_params=pltpu.CompilerParams(collective_id=0),
    )(x, Win0, Wout0, Win1, Wout1, Win2, Wout2)
